# baseline (speedup 1.0000x reference)
import functools

import jax
import jax.numpy as jnp
from jax import lax
from jax.experimental import pallas as pl
from jax.experimental.pallas import tpu as pltpu
from jax.experimental.pallas import tpu_sc as plsc

N = 10000
E = 320000
NPAD = 10240
ER = 2560
EPAD = ER * 128
F_IN = 128
F_HID = 256
F_OUT = 64
BN = 1024
GRID = NPAD // BN
RPT = NPAD // 16

_mesh = functools.partial(
    plsc.VectorSubcoreMesh, core_axis_name="c", subcore_axis_name="s",
    num_cores=2, num_subcores=16)


def _deg_body(srcr, dstr, zeros_h, ones_h, degs,
              idx_v, ones_v, acc, sem):
    cid = lax.axis_index("c")
    sid = lax.axis_index("s")
    pltpu.sync_copy(ones_h, ones_v)
    pltpu.sync_copy(zeros_h, acc.at[pl.ds(sid * RPT, RPT)])
    plsc.subcore_barrier()
    row_base = sid * (ER // 16)

    def chunk(c, carry):
        r0 = row_base + c * 8

        @pl.when(cid == 0)
        def _():
            pltpu.sync_copy(srcr.at[pl.ds(r0, 8)], idx_v)

        @pl.when(cid == 1)
        def _():
            pltpu.sync_copy(dstr.at[pl.ds(r0, 8)], idx_v)

        pend = [pltpu.async_copy(ones_v, acc.at[idx_v.at[j]], sem, add=True)
                for j in range(8)]
        for s in pend:
            s.wait()
        return carry

    lax.fori_loop(0, (ER // 16) // 8, chunk, 0)
    plsc.subcore_barrier()
    sl = pl.ds(sid * RPT, RPT)
    pltpu.sync_copy(acc.at[sl], degs.at[cid, sl])


def _make_deg(interpret=False):
    return pl.kernel(
        _deg_body,
        out_type=jax.ShapeDtypeStruct((2, NPAD, 128), jnp.float32),
        mesh=_mesh(),
        interpret=interpret,
        scratch_types=[
            pltpu.VMEM((8, 128), jnp.int32),
            pltpu.VMEM((128, 128), jnp.float32),
            pltpu.VMEM_SHARED((NPAD, 128), jnp.float32),
            pltpu.SemaphoreType.DMA,
        ],
    )


_deg = _make_deg()


def _make_agg(D, mode, interpret=False):
    nrows = (ER // 32) if mode == "edge" else (ER // 16)

    def body(*args):
        if mode == "edge":
            (table, srcr, dstr, zeros_h, out,
             src_a, dst_a, msg0, msg1, acc, sg0, sg1, ss) = args
        else:
            (t0, t1, srcr, dstr, zeros_h, out,
             src_a, dst_a, msg0, msg1, acc, sg0, sg1, ss) = args
        cid = lax.axis_index("c")
        sid = lax.axis_index("s")
        if mode == "edge":
            row_base = (cid * 16 + sid) * nrows
        else:
            row_base = sid * nrows
        pltpu.sync_copy(zeros_h, acc.at[pl.ds(sid * RPT, RPT)])
        plsc.subcore_barrier()

        def gather(j, buf, sem):
            for k in range(4):
                idx = src_a.at[j, pl.ds(32 * k, 32)]
                sub = buf.at[pl.ds(32 * k, 32)]
                if mode == "edge":
                    pltpu.async_copy(table.at[idx], sub, sem)
                else:
                    @pl.when(cid == 0)
                    def _():
                        pltpu.async_copy(t0.at[idx], sub, sem)

                    @pl.when(cid == 1)
                    def _():
                        pltpu.async_copy(t1.at[idx], sub, sem)
            dummy = table if mode == "edge" else t0
            return pltpu.make_async_copy(dummy.at[pl.ds(0, 128)], buf, sem)

        def chunk(c, carry):
            r0 = row_base + c * 8
            pltpu.sync_copy(srcr.at[pl.ds(r0, 8)], src_a)
            pltpu.sync_copy(dstr.at[pl.ds(r0, 8)], dst_a)
            for p in range(4):
                j0 = 2 * p
                j1 = j0 + 1
                g0 = gather(j0, msg0, sg0)
                g1 = gather(j1, msg1, sg1)
                g0.wait()
                s0 = pltpu.async_copy(msg0, acc.at[dst_a.at[j0]], ss,
                                      add=True)
                g1.wait()
                s1 = pltpu.async_copy(msg1, acc.at[dst_a.at[j1]], ss,
                                      add=True)
                s0.wait()
                s1.wait()
            return carry

        lax.fori_loop(0, nrows // 8, chunk, 0)
        plsc.subcore_barrier()
        sl = pl.ds(sid * RPT, RPT)
        pltpu.sync_copy(acc.at[sl], out.at[cid, sl])

    return pl.kernel(
        body,
        out_type=jax.ShapeDtypeStruct((2, NPAD, D), jnp.float32),
        mesh=_mesh(),
        interpret=interpret,
        scratch_types=[
            pltpu.VMEM((8, 128), jnp.int32),
            pltpu.VMEM((8, 128), jnp.int32),
            pltpu.VMEM((128, D), jnp.float32),
            pltpu.VMEM((128, D), jnp.float32),
            pltpu.VMEM_SHARED((NPAD, D), jnp.float32),
            pltpu.SemaphoreType.DMA,
            pltpu.SemaphoreType.DMA,
            pltpu.SemaphoreType.DMA,
        ],
    )


_agg1 = _make_agg(128, "edge")
_agg2 = _make_agg(128, "col")
_agg3 = _make_agg(128, "edge")


def _norms(degs_ref):
    no = lax.rsqrt(jnp.maximum(jnp.max(degs_ref[0], axis=1), 1.0))
    ni = lax.rsqrt(jnp.maximum(jnp.max(degs_ref[1], axis=1), 1.0))
    return no, ni


def _b1_body(degs_ref, x_ref, o_ref):
    no, _ = _norms(degs_ref)
    o_ref[...] = x_ref[...] * no[:, None]


def _b2_body(a1_ref, degs_ref, w1_ref, b1_ref, w2_ref,
             oa_ref, ob_ref):
    no, ni = _norms(degs_ref)
    agg = a1_ref[0] + a1_ref[1]
    h = jnp.dot(agg, w1_ref[...], preferred_element_type=jnp.float32)
    h = jnp.maximum(h * ni[:, None] + b1_ref[...], 0.0)
    t = h * no[:, None]
    oa_ref[...] = jnp.dot(t, w2_ref[:, :128], preferred_element_type=jnp.float32)
    ob_ref[...] = jnp.dot(t, w2_ref[:, 128:], preferred_element_type=jnp.float32)


def _b3_body(a2_ref, degs_ref, b2_ref, w3_ref, o_ref):
    no, ni = _norms(degs_ref)
    agg = jnp.concatenate([a2_ref[0], a2_ref[1]], axis=1)
    h = jnp.maximum(agg * ni[:, None] + b2_ref[...], 0.0)
    t = h * no[:, None]
    o_ref[...] = jnp.dot(t, w3_ref[...], preferred_element_type=jnp.float32)


def _b4_body(a3_ref, degs_ref, b3_ref, o_ref):
    _, ni = _norms(degs_ref)
    agg = (a3_ref[0] + a3_ref[1])[:, :64]
    o_ref[...] = agg * ni[:, None] + b3_ref[...]


def _deg_spec():
    return pl.BlockSpec((2, BN, 128), lambda i: (0, i, 0))


def _b1(degs, xpad):
    return pl.pallas_call(
        _b1_body, grid=(GRID,),
        in_specs=[_deg_spec(), pl.BlockSpec((BN, 128), lambda i: (i, 0))],
        out_specs=pl.BlockSpec((BN, 128), lambda i: (i, 0)),
        out_shape=jax.ShapeDtypeStruct((NPAD, 128), jnp.float32),
    )(degs, xpad)


def _b2(agg1, degs, W1, b1, W2):
    return pl.pallas_call(
        _b2_body, grid=(GRID,),
        in_specs=[
            pl.BlockSpec((2, BN, 128), lambda i: (0, i, 0)),
            _deg_spec(),
            pl.BlockSpec((128, 256), lambda i: (0, 0)),
            pl.BlockSpec((1, 256), lambda i: (0, 0)),
            pl.BlockSpec((256, 256), lambda i: (0, 0)),
        ],
        out_specs=[pl.BlockSpec((BN, 128), lambda i: (i, 0)),
                   pl.BlockSpec((BN, 128), lambda i: (i, 0))],
        out_shape=[jax.ShapeDtypeStruct((NPAD, 128), jnp.float32),
                   jax.ShapeDtypeStruct((NPAD, 128), jnp.float32)],
    )(agg1, degs, W1, b1, W2)


def _b3(agg2, degs, b2, W3):
    return pl.pallas_call(
        _b3_body, grid=(GRID,),
        in_specs=[
            pl.BlockSpec((2, BN, 128), lambda i: (0, i, 0)),
            _deg_spec(),
            pl.BlockSpec((1, 256), lambda i: (0, 0)),
            pl.BlockSpec((256, 128), lambda i: (0, 0)),
        ],
        out_specs=pl.BlockSpec((BN, 128), lambda i: (i, 0)),
        out_shape=jax.ShapeDtypeStruct((NPAD, 128), jnp.float32),
    )(agg2, degs, b2, W3)


def _b4(agg3, degs, b3):
    return pl.pallas_call(
        _b4_body, grid=(GRID,),
        in_specs=[
            pl.BlockSpec((2, BN, 128), lambda i: (0, i, 0)),
            _deg_spec(),
            pl.BlockSpec((1, 64), lambda i: (0, 0)),
        ],
        out_specs=pl.BlockSpec((BN, 64), lambda i: (i, 0)),
        out_shape=jax.ShapeDtypeStruct((NPAD, 64), jnp.float32),
    )(agg3, degs, b3)


def kernel(g, features, W1, b1, W2, b2, W3, b3):
    pad = N + jnp.arange(EPAD - E, dtype=jnp.int32) % (NPAD - N)
    srcr = jnp.concatenate([g[0].astype(jnp.int32), pad]).reshape(ER, 128)
    dstr = jnp.concatenate([g[1].astype(jnp.int32), pad]).reshape(ER, 128)
    xpad = jnp.pad(features, ((0, NPAD - N), (0, 0)))
    ones128 = jnp.ones((128, 128), jnp.float32)
    zeros128 = jnp.zeros((RPT, 128), jnp.float32)
    W3p = jnp.pad(W3, ((0, 0), (0, 64)))

    degs = _deg(srcr, dstr, zeros128, ones128)
    x1 = _b1(degs, xpad)
    agg1 = _agg1(x1, srcr, dstr, zeros128)
    hw2a, hw2b = _b2(agg1, degs, W1, b1[None, :], W2)
    agg2 = _agg2(hw2a, hw2b, srcr, dstr, zeros128)
    hw3 = _b3(agg2, degs, b2[None, :], W3p)
    agg3 = _agg3(hw3, srcr, dstr, zeros128)
    out = _b4(agg3, degs, b3[None, :])
    return out[:N]


# speedup vs baseline: 1.0849x; 1.0849x over previous
import functools

import jax
import jax.numpy as jnp
from jax import lax
from jax.experimental import pallas as pl
from jax.experimental.pallas import tpu as pltpu
from jax.experimental.pallas import tpu_sc as plsc

N = 10000
E = 320000
NPAD = 10240
ER = 2560
EPAD = ER * 128
F_IN = 128
F_HID = 256
F_OUT = 64
BN = 1024
GRID = NPAD // BN
RPT = NPAD // 16

_mesh = functools.partial(
    plsc.VectorSubcoreMesh, core_axis_name="c", subcore_axis_name="s",
    num_cores=2, num_subcores=16)


_HALF = NPAD // 2


def _deg_body(srcr, dstr, zeros_h, out, idx_v, acc, res_v):
    cid = lax.axis_index("c")
    sid = lax.axis_index("s")
    lane_off = lax.iota(jnp.int32, 16) * _HALF
    ones_v = jnp.ones((16,), jnp.float32)
    nrows = ER // 16
    row_base = sid * nrows

    for half in range(2):
        lo = half * _HALF
        pltpu.sync_copy(zeros_h, acc)

        def chunk(c, carry):
            r0 = row_base + c * 8

            @pl.when(cid == 0)
            def _():
                pltpu.sync_copy(srcr.at[pl.ds(r0, 8)], idx_v)

            @pl.when(cid == 1)
            def _():
                pltpu.sync_copy(dstr.at[pl.ds(r0, 8)], idx_v)

            for j in range(8):
                for k in range(8):
                    v = idx_v[j, pl.ds(16 * k, 16)]
                    m = (v >= lo) & (v < lo + _HALF)
                    addr = jnp.where(m, v - lo, 0) + lane_off
                    plsc.addupdate_scatter(acc, [addr], ones_v, mask=m)
            return carry

        lax.fori_loop(0, nrows // 8, chunk, 0)

        def red(j2, carry):
            s = acc[pl.ds(j2 * 16, 16)]
            for l in range(1, 16):
                s = s + acc[pl.ds(l * _HALF + j2 * 16, 16)]
            res_v[pl.ds(j2 * 16, 16)] = s
            return carry

        lax.fori_loop(0, _HALF // 16, red, 0)
        pltpu.sync_copy(res_v, out.at[cid, sid, pl.ds(lo, _HALF)])


def _make_deg(interpret=False):
    return pl.kernel(
        _deg_body,
        out_type=jax.ShapeDtypeStruct((2, 16, NPAD), jnp.float32),
        mesh=_mesh(),
        interpret=interpret,
        compiler_params=pltpu.CompilerParams(needs_layout_passes=False),
        scratch_types=[
            pltpu.VMEM((8, 128), jnp.int32),
            pltpu.VMEM((16 * _HALF,), jnp.float32),
            pltpu.VMEM((_HALF,), jnp.float32),
        ],
    )


_deg = _make_deg()


def _make_agg(D, mode, interpret=False):
    nrows = (ER // 32) if mode == "edge" else (ER // 16)

    def body(*args):
        if mode == "edge":
            (table, srcr, dstr, zeros_h, out,
             src_a, dst_a, msg0, msg1, acc, sg0, sg1, ss) = args
        else:
            (t0, t1, srcr, dstr, zeros_h, out,
             src_a, dst_a, msg0, msg1, acc, sg0, sg1, ss) = args
        cid = lax.axis_index("c")
        sid = lax.axis_index("s")
        if mode == "edge":
            row_base = (cid * 16 + sid) * nrows
        else:
            row_base = sid * nrows
        pltpu.sync_copy(zeros_h, acc.at[pl.ds(sid * RPT, RPT)])
        plsc.subcore_barrier()

        def gather(j, buf, sem):
            for k in range(4):
                idx = src_a.at[j, pl.ds(32 * k, 32)]
                sub = buf.at[pl.ds(32 * k, 32)]
                if mode == "edge":
                    pltpu.async_copy(table.at[idx], sub, sem)
                else:
                    @pl.when(cid == 0)
                    def _():
                        pltpu.async_copy(t0.at[idx], sub, sem)

                    @pl.when(cid == 1)
                    def _():
                        pltpu.async_copy(t1.at[idx], sub, sem)
            dummy = table if mode == "edge" else t0
            return pltpu.make_async_copy(dummy.at[pl.ds(0, 128)], buf, sem)

        def chunk(c, carry):
            r0 = row_base + c * 8
            pltpu.sync_copy(srcr.at[pl.ds(r0, 8)], src_a)
            pltpu.sync_copy(dstr.at[pl.ds(r0, 8)], dst_a)
            for p in range(4):
                j0 = 2 * p
                j1 = j0 + 1
                g0 = gather(j0, msg0, sg0)
                g1 = gather(j1, msg1, sg1)
                g0.wait()
                s0 = pltpu.async_copy(msg0, acc.at[dst_a.at[j0]], ss,
                                      add=True)
                g1.wait()
                s1 = pltpu.async_copy(msg1, acc.at[dst_a.at[j1]], ss,
                                      add=True)
                s0.wait()
                s1.wait()
            return carry

        lax.fori_loop(0, nrows // 8, chunk, 0)
        plsc.subcore_barrier()
        sl = pl.ds(sid * RPT, RPT)
        pltpu.sync_copy(acc.at[sl], out.at[cid, sl])

    return pl.kernel(
        body,
        out_type=jax.ShapeDtypeStruct((2, NPAD, D), jnp.float32),
        mesh=_mesh(),
        interpret=interpret,
        scratch_types=[
            pltpu.VMEM((8, 128), jnp.int32),
            pltpu.VMEM((8, 128), jnp.int32),
            pltpu.VMEM((128, D), jnp.float32),
            pltpu.VMEM((128, D), jnp.float32),
            pltpu.VMEM_SHARED((NPAD, D), jnp.float32),
            pltpu.SemaphoreType.DMA,
            pltpu.SemaphoreType.DMA,
            pltpu.SemaphoreType.DMA,
        ],
    )


_agg1 = _make_agg(128, "edge")
_agg2 = _make_agg(128, "col")
_agg3 = _make_agg(128, "edge")


def _norms(degs_ref):
    no = lax.rsqrt(jnp.maximum(jnp.sum(degs_ref[0], axis=0), 1.0))
    ni = lax.rsqrt(jnp.maximum(jnp.sum(degs_ref[1], axis=0), 1.0))
    return no, ni


def _b1_body(degs_ref, x_ref, o_ref):
    no, _ = _norms(degs_ref)
    o_ref[...] = x_ref[...] * no[:, None]


def _b2_body(a1_ref, degs_ref, w1_ref, b1_ref, w2_ref,
             oa_ref, ob_ref):
    no, ni = _norms(degs_ref)
    agg = a1_ref[0] + a1_ref[1]
    h = jnp.dot(agg, w1_ref[...], preferred_element_type=jnp.float32)
    h = jnp.maximum(h * ni[:, None] + b1_ref[...], 0.0)
    t = h * no[:, None]
    oa_ref[...] = jnp.dot(t, w2_ref[:, :128], preferred_element_type=jnp.float32)
    ob_ref[...] = jnp.dot(t, w2_ref[:, 128:], preferred_element_type=jnp.float32)


def _b3_body(a2_ref, degs_ref, b2_ref, w3_ref, o_ref):
    no, ni = _norms(degs_ref)
    agg = jnp.concatenate([a2_ref[0], a2_ref[1]], axis=1)
    h = jnp.maximum(agg * ni[:, None] + b2_ref[...], 0.0)
    t = h * no[:, None]
    o_ref[...] = jnp.dot(t, w3_ref[...], preferred_element_type=jnp.float32)


def _b4_body(a3_ref, degs_ref, b3_ref, o_ref):
    _, ni = _norms(degs_ref)
    agg = (a3_ref[0] + a3_ref[1])[:, :64]
    o_ref[...] = agg * ni[:, None] + b3_ref[...]


def _deg_spec():
    return pl.BlockSpec((2, 16, BN), lambda i: (0, 0, i))


def _b1(degs, xpad):
    return pl.pallas_call(
        _b1_body, grid=(GRID,),
        in_specs=[_deg_spec(), pl.BlockSpec((BN, 128), lambda i: (i, 0))],
        out_specs=pl.BlockSpec((BN, 128), lambda i: (i, 0)),
        out_shape=jax.ShapeDtypeStruct((NPAD, 128), jnp.float32),
    )(degs, xpad)


def _b2(agg1, degs, W1, b1, W2):
    return pl.pallas_call(
        _b2_body, grid=(GRID,),
        in_specs=[
            pl.BlockSpec((2, BN, 128), lambda i: (0, i, 0)),
            _deg_spec(),
            pl.BlockSpec((128, 256), lambda i: (0, 0)),
            pl.BlockSpec((1, 256), lambda i: (0, 0)),
            pl.BlockSpec((256, 256), lambda i: (0, 0)),
        ],
        out_specs=[pl.BlockSpec((BN, 128), lambda i: (i, 0)),
                   pl.BlockSpec((BN, 128), lambda i: (i, 0))],
        out_shape=[jax.ShapeDtypeStruct((NPAD, 128), jnp.float32),
                   jax.ShapeDtypeStruct((NPAD, 128), jnp.float32)],
    )(agg1, degs, W1, b1, W2)


def _b3(agg2, degs, b2, W3):
    return pl.pallas_call(
        _b3_body, grid=(GRID,),
        in_specs=[
            pl.BlockSpec((2, BN, 128), lambda i: (0, i, 0)),
            _deg_spec(),
            pl.BlockSpec((1, 256), lambda i: (0, 0)),
            pl.BlockSpec((256, 128), lambda i: (0, 0)),
        ],
        out_specs=pl.BlockSpec((BN, 128), lambda i: (i, 0)),
        out_shape=jax.ShapeDtypeStruct((NPAD, 128), jnp.float32),
    )(agg2, degs, b2, W3)


def _b4(agg3, degs, b3):
    return pl.pallas_call(
        _b4_body, grid=(GRID,),
        in_specs=[
            pl.BlockSpec((2, BN, 128), lambda i: (0, i, 0)),
            _deg_spec(),
            pl.BlockSpec((1, 64), lambda i: (0, 0)),
        ],
        out_specs=pl.BlockSpec((BN, 64), lambda i: (i, 0)),
        out_shape=jax.ShapeDtypeStruct((NPAD, 64), jnp.float32),
    )(agg3, degs, b3)


def kernel(g, features, W1, b1, W2, b2, W3, b3):
    pad = N + jnp.arange(EPAD - E, dtype=jnp.int32) % (NPAD - N)
    srcr = jnp.concatenate([g[0].astype(jnp.int32), pad]).reshape(ER, 128)
    dstr = jnp.concatenate([g[1].astype(jnp.int32), pad]).reshape(ER, 128)
    xpad = jnp.pad(features, ((0, NPAD - N), (0, 0)))
    zeros128 = jnp.zeros((RPT, 128), jnp.float32)
    zeros_deg = jnp.zeros((16 * _HALF,), jnp.float32)
    W3p = jnp.pad(W3, ((0, 0), (0, 64)))

    degs = _deg(srcr, dstr, zeros_deg)
    x1 = _b1(degs, xpad)
    agg1 = _agg1(x1, srcr, dstr, zeros128)
    hw2a, hw2b = _b2(agg1, degs, W1, b1[None, :], W2)
    agg2 = _agg2(hw2a, hw2b, srcr, dstr, zeros128)
    hw3 = _b3(agg2, degs, b2[None, :], W3p)
    agg3 = _agg3(hw3, srcr, dstr, zeros128)
    out = _b4(agg3, degs, b3[None, :])
    return out[:N]


# speedup vs baseline: 1.0851x; 1.0002x over previous
import functools

import jax
import jax.numpy as jnp
from jax import lax
from jax.experimental import pallas as pl
from jax.experimental.pallas import tpu as pltpu
from jax.experimental.pallas import tpu_sc as plsc

N = 10000
E = 320000
NPAD = 10240
ER = 2560
EPAD = ER * 128
F_IN = 128
F_HID = 256
F_OUT = 64
BN = 1024
GRID = NPAD // BN
RPT = NPAD // 16

_mesh = functools.partial(
    plsc.VectorSubcoreMesh, core_axis_name="c", subcore_axis_name="s",
    num_cores=2, num_subcores=16)


_HALF = NPAD // 2


def _deg_body(srcr, dstr, zeros_h, out, idx_v, acc, res_v):
    cid = lax.axis_index("c")
    sid = lax.axis_index("s")
    lane_off = lax.iota(jnp.int32, 16) * _HALF
    ones_v = jnp.ones((16,), jnp.float32)
    nrows = ER // 16
    row_base = sid * nrows

    for half in range(2):
        lo = half * _HALF
        pltpu.sync_copy(zeros_h, acc)

        def chunk(c, carry):
            r0 = row_base + c * 8

            @pl.when(cid == 0)
            def _():
                pltpu.sync_copy(srcr.at[pl.ds(r0, 8)], idx_v)

            @pl.when(cid == 1)
            def _():
                pltpu.sync_copy(dstr.at[pl.ds(r0, 8)], idx_v)

            for j in range(8):
                for k in range(8):
                    v = idx_v[j, pl.ds(16 * k, 16)]
                    m = (v >= lo) & (v < lo + _HALF)
                    addr = jnp.where(m, v - lo, 0) + lane_off
                    plsc.addupdate_scatter(acc, [addr], ones_v, mask=m)
            return carry

        lax.fori_loop(0, nrows // 8, chunk, 0)

        def red(j2, carry):
            s = acc[pl.ds(j2 * 16, 16)]
            for l in range(1, 16):
                s = s + acc[pl.ds(l * _HALF + j2 * 16, 16)]
            res_v[pl.ds(j2 * 16, 16)] = s
            return carry

        lax.fori_loop(0, _HALF // 16, red, 0)
        pltpu.sync_copy(res_v, out.at[cid, sid, pl.ds(lo, _HALF)])


def _make_deg(interpret=False):
    return pl.kernel(
        _deg_body,
        out_type=jax.ShapeDtypeStruct((2, 16, NPAD), jnp.float32),
        mesh=_mesh(),
        interpret=interpret,
        compiler_params=pltpu.CompilerParams(needs_layout_passes=False),
        scratch_types=[
            pltpu.VMEM((8, 128), jnp.int32),
            pltpu.VMEM((16 * _HALF,), jnp.float32),
            pltpu.VMEM((_HALF,), jnp.float32),
        ],
    )


_deg = _make_deg()


def _make_agg(D, mode, interpret=False):
    nrows = (ER // 32) if mode == "edge" else (ER // 16)

    def body(*args):
        if mode == "edge":
            (table, srcr, dstr, zeros_h, out,
             src_a, dst_a, msg0, msg1, acc, sg0, sg1, ss) = args
        else:
            (t0, t1, srcr, dstr, zeros_h, out,
             src_a, dst_a, msg0, msg1, acc, sg0, sg1, ss) = args
        cid = lax.axis_index("c")
        sid = lax.axis_index("s")
        if mode == "edge":
            row_base = (cid * 16 + sid) * nrows
        else:
            row_base = sid * nrows
        pltpu.sync_copy(zeros_h, acc.at[pl.ds(sid * RPT, RPT)])
        plsc.subcore_barrier()

        def gather(j, buf, sem):
            for k in range(8):
                idx = src_a.at[j, pl.ds(16 * k, 16)]
                sub = buf.at[pl.ds(16 * k, 16)]
                if mode == "edge":
                    pltpu.async_copy(table.at[idx], sub, sem)
                else:
                    @pl.when(cid == 0)
                    def _():
                        pltpu.async_copy(t0.at[idx], sub, sem)

                    @pl.when(cid == 1)
                    def _():
                        pltpu.async_copy(t1.at[idx], sub, sem)
            dummy = table if mode == "edge" else t0
            return pltpu.make_async_copy(dummy.at[pl.ds(0, 128)], buf, sem)

        def chunk(c, carry):
            r0 = row_base + c * 8
            pltpu.sync_copy(srcr.at[pl.ds(r0, 8)], src_a)
            pltpu.sync_copy(dstr.at[pl.ds(r0, 8)], dst_a)
            for p in range(4):
                j0 = 2 * p
                j1 = j0 + 1
                g0 = gather(j0, msg0, sg0)
                g1 = gather(j1, msg1, sg1)
                g0.wait()
                s0 = pltpu.async_copy(msg0, acc.at[dst_a.at[j0]], ss,
                                      add=True)
                g1.wait()
                s1 = pltpu.async_copy(msg1, acc.at[dst_a.at[j1]], ss,
                                      add=True)
                s0.wait()
                s1.wait()
            return carry

        lax.fori_loop(0, nrows // 8, chunk, 0)
        plsc.subcore_barrier()
        sl = pl.ds(sid * RPT, RPT)
        pltpu.sync_copy(acc.at[sl], out.at[cid, sl])

    return pl.kernel(
        body,
        out_type=jax.ShapeDtypeStruct((2, NPAD, D), jnp.float32),
        mesh=_mesh(),
        interpret=interpret,
        scratch_types=[
            pltpu.VMEM((8, 128), jnp.int32),
            pltpu.VMEM((8, 128), jnp.int32),
            pltpu.VMEM((128, D), jnp.float32),
            pltpu.VMEM((128, D), jnp.float32),
            pltpu.VMEM_SHARED((NPAD, D), jnp.float32),
            pltpu.SemaphoreType.DMA,
            pltpu.SemaphoreType.DMA,
            pltpu.SemaphoreType.DMA,
        ],
    )


_agg1 = _make_agg(128, "edge")
_agg2 = _make_agg(128, "col")
_agg3 = _make_agg(128, "edge")


def _norms(degs_ref):
    no = lax.rsqrt(jnp.maximum(jnp.sum(degs_ref[0], axis=0), 1.0))
    ni = lax.rsqrt(jnp.maximum(jnp.sum(degs_ref[1], axis=0), 1.0))
    return no, ni


def _b1_body(degs_ref, x_ref, o_ref):
    no, _ = _norms(degs_ref)
    o_ref[...] = x_ref[...] * no[:, None]


def _b2_body(a1_ref, degs_ref, w1_ref, b1_ref, w2_ref,
             oa_ref, ob_ref):
    no, ni = _norms(degs_ref)
    agg = a1_ref[0] + a1_ref[1]
    h = jnp.dot(agg, w1_ref[...], preferred_element_type=jnp.float32)
    h = jnp.maximum(h * ni[:, None] + b1_ref[...], 0.0)
    t = h * no[:, None]
    oa_ref[...] = jnp.dot(t, w2_ref[:, :128], preferred_element_type=jnp.float32)
    ob_ref[...] = jnp.dot(t, w2_ref[:, 128:], preferred_element_type=jnp.float32)


def _b3_body(a2_ref, degs_ref, b2_ref, w3_ref, o_ref):
    no, ni = _norms(degs_ref)
    agg = jnp.concatenate([a2_ref[0], a2_ref[1]], axis=1)
    h = jnp.maximum(agg * ni[:, None] + b2_ref[...], 0.0)
    t = h * no[:, None]
    o_ref[...] = jnp.dot(t, w3_ref[...], preferred_element_type=jnp.float32)


def _b4_body(a3_ref, degs_ref, b3_ref, o_ref):
    _, ni = _norms(degs_ref)
    agg = (a3_ref[0] + a3_ref[1])[:, :64]
    o_ref[...] = agg * ni[:, None] + b3_ref[...]


def _deg_spec():
    return pl.BlockSpec((2, 16, BN), lambda i: (0, 0, i))


def _b1(degs, xpad):
    return pl.pallas_call(
        _b1_body, grid=(GRID,),
        in_specs=[_deg_spec(), pl.BlockSpec((BN, 128), lambda i: (i, 0))],
        out_specs=pl.BlockSpec((BN, 128), lambda i: (i, 0)),
        out_shape=jax.ShapeDtypeStruct((NPAD, 128), jnp.float32),
    )(degs, xpad)


def _b2(agg1, degs, W1, b1, W2):
    return pl.pallas_call(
        _b2_body, grid=(GRID,),
        in_specs=[
            pl.BlockSpec((2, BN, 128), lambda i: (0, i, 0)),
            _deg_spec(),
            pl.BlockSpec((128, 256), lambda i: (0, 0)),
            pl.BlockSpec((1, 256), lambda i: (0, 0)),
            pl.BlockSpec((256, 256), lambda i: (0, 0)),
        ],
        out_specs=[pl.BlockSpec((BN, 128), lambda i: (i, 0)),
                   pl.BlockSpec((BN, 128), lambda i: (i, 0))],
        out_shape=[jax.ShapeDtypeStruct((NPAD, 128), jnp.float32),
                   jax.ShapeDtypeStruct((NPAD, 128), jnp.float32)],
    )(agg1, degs, W1, b1, W2)


def _b3(agg2, degs, b2, W3):
    return pl.pallas_call(
        _b3_body, grid=(GRID,),
        in_specs=[
            pl.BlockSpec((2, BN, 128), lambda i: (0, i, 0)),
            _deg_spec(),
            pl.BlockSpec((1, 256), lambda i: (0, 0)),
            pl.BlockSpec((256, 128), lambda i: (0, 0)),
        ],
        out_specs=pl.BlockSpec((BN, 128), lambda i: (i, 0)),
        out_shape=jax.ShapeDtypeStruct((NPAD, 128), jnp.float32),
    )(agg2, degs, b2, W3)


def _b4(agg3, degs, b3):
    return pl.pallas_call(
        _b4_body, grid=(GRID,),
        in_specs=[
            pl.BlockSpec((2, BN, 128), lambda i: (0, i, 0)),
            _deg_spec(),
            pl.BlockSpec((1, 64), lambda i: (0, 0)),
        ],
        out_specs=pl.BlockSpec((BN, 64), lambda i: (i, 0)),
        out_shape=jax.ShapeDtypeStruct((NPAD, 64), jnp.float32),
    )(agg3, degs, b3)


def kernel(g, features, W1, b1, W2, b2, W3, b3):
    pad = N + jnp.arange(EPAD - E, dtype=jnp.int32) % (NPAD - N)
    srcr = jnp.concatenate([g[0].astype(jnp.int32), pad]).reshape(ER, 128)
    dstr = jnp.concatenate([g[1].astype(jnp.int32), pad]).reshape(ER, 128)
    xpad = jnp.pad(features, ((0, NPAD - N), (0, 0)))
    zeros128 = jnp.zeros((RPT, 128), jnp.float32)
    zeros_deg = jnp.zeros((16 * _HALF,), jnp.float32)
    W3p = jnp.pad(W3, ((0, 0), (0, 64)))

    degs = _deg(srcr, dstr, zeros_deg)
    x1 = _b1(degs, xpad)
    agg1 = _agg1(x1, srcr, dstr, zeros128)
    hw2a, hw2b = _b2(agg1, degs, W1, b1[None, :], W2)
    agg2 = _agg2(hw2a, hw2b, srcr, dstr, zeros128)
    hw3 = _b3(agg2, degs, b2[None, :], W3p)
    agg3 = _agg3(hw3, srcr, dstr, zeros128)
    out = _b4(agg3, degs, b3[None, :])
    return out[:N]
